# Initial kernel scaffold; baseline (speedup 1.0000x reference)
import jax
import jax.numpy as jnp
from jax.experimental import pallas as pl


def kernel(idx, dists, features):
    raise NotImplementedError("write your pallas kernel here")


# speedup vs baseline: 5.2767x; 5.2767x over previous
import functools

import jax
import jax.numpy as jnp
from jax import lax
from jax.experimental import pallas as pl
from jax.experimental.pallas import tpu as pltpu
from jax.experimental.pallas import tpu_sc as plsc

B, H, W, K, P, C = 4, 224, 224, 8, 100000, 32
N = B * H * W
HW = H * W
NC, NS = 2, 16
NW = NC * NS
NPW = N // NW
Q = 128
NCHUNK = NPW // Q
WPB = HW // NPW


def _sc_render(idx_hbm, dists_hbm, feat_hbm, img_hbm, dtr_hbm,
               idx_v, dists_v, w_v, dtr_v, rows_v, out_v, gsem):
    cid = lax.axis_index("c")
    sid = lax.axis_index("s")
    wid = sid * NC + cid
    lanes = lax.iota(jnp.int32, 16)
    col8 = lanes * 8

    def chunk_body(ci, _):
        cidx = wid * NCHUNK + ci
        pltpu.sync_copy(idx_hbm.at[cidx], idx_v)
        pltpu.sync_copy(dists_hbm.at[cidx], dists_v)

        copies = []
        for g in range(8):
            copies.append(
                pltpu.async_copy(feat_hbm.at[idx_v.at[g]],
                                 rows_v.at[pl.ds(g * 128, 128)], gsem))

        for g in range(8):
            rowv = jnp.full((16,), g, jnp.int32)
            trans = jnp.full((16,), 1.0, jnp.float32)
            for k in range(K):
                d = plsc.load_gather(dists_v, [rowv, col8 + k])
                dtr_v[k, pl.ds(ci * Q + g * 16, 16)] = d
                alpha = jnp.clip(1.0 - d, 0.0, 1.0)
                plsc.store_scatter(w_v, [rowv, col8 + k], alpha * trans)
                trans = trans * (1.0 - alpha)

        for cp in copies:
            cp.wait()

        def grp_body(g, carry):
            gv = jnp.full((16,), g, jnp.int32)
            rowbase = g * 128 + col8
            accs = [jnp.zeros((16,), jnp.float32) for _ in range(C)]
            for k in range(K):
                wk = plsc.load_gather(w_v, [gv, col8 + k])
                ridx = rowbase + k
                for c in range(C):
                    cv = jnp.full((16,), c, jnp.int32)
                    val = plsc.load_gather(rows_v, [ridx, cv])
                    accs[c] = accs[c] + wk * val
            prow = g * 16 + lanes
            for c in range(C):
                cv = jnp.full((16,), c, jnp.int32)
                plsc.store_scatter(out_v, [prow, cv], accs[c])
            return carry

        lax.fori_loop(0, 8, grp_body, None)

        pltpu.sync_copy(out_v, img_hbm.at[cidx])
        return _

    lax.fori_loop(0, NCHUNK, chunk_body, None)

    b = wid // WPB
    j = wid % WPB
    r0 = pl.multiple_of(b * K, 8)
    c0 = pl.multiple_of(j * NPW, 128)
    pltpu.sync_copy(dtr_v, dtr_hbm.at[pl.ds(r0, K), pl.ds(c0, NPW)])


_sc_render_call = functools.partial(
    pl.kernel,
    out_type=[jax.ShapeDtypeStruct((NW * NCHUNK, Q, C), jnp.float32),
              jax.ShapeDtypeStruct((B * K, HW), jnp.float32)],
    mesh=plsc.VectorSubcoreMesh(core_axis_name="c", subcore_axis_name="s"),
    scratch_types=[
        pltpu.VMEM((K, 128), jnp.int32),
        pltpu.VMEM((K, 128), jnp.float32),
        pltpu.VMEM((K, 128), jnp.float32),
        pltpu.VMEM((K, NPW), jnp.float32),
        pltpu.VMEM((Q * K, C), jnp.float32),
        pltpu.VMEM((Q, C), jnp.float32),
        pltpu.SemaphoreType.DMA,
    ],
    compiler_params=pltpu.CompilerParams(needs_layout_passes=False,
                                         use_tc_tiling_on_sc=False),
)(_sc_render)


def kernel(idx, dists, features):
    idx3 = idx.reshape(NW * NCHUNK, K, 128)
    dists3 = dists.reshape(NW * NCHUNK, K, 128)
    img3, dtr = _sc_render_call(idx3, dists3, features)
    images = img3.reshape(B, H, W, C)
    dists2 = dtr.reshape(B, K, H, W)
    return images, dists2


# speedup vs baseline: 13.9083x; 2.6358x over previous
import functools

import jax
import jax.numpy as jnp
from jax import lax
from jax.experimental import pallas as pl
from jax.experimental.pallas import tpu as pltpu
from jax.experimental.pallas import tpu_sc as plsc

B, H, W, K, P, C = 4, 224, 224, 8, 100000, 32
N = B * H * W
HW = H * W
NC, NS = 2, 16
NW = NC * NS
NPW = N // NW
Q = 128
NCHUNK = NPW // Q
WPB = HW // NPW
CH = C // 2


def _sc_render(idx_hbm, dists_hbm, feat_hbm, img_hbm, dtr_hbm,
               idx_v0, idx_v1, dists_v0, dists_v1, wt_v0, wt_v1,
               dtr_v0, dtr_v1, rows_v0, rows_v1, out_v0, out_v1,
               isem0, isem1, gsem0, gsem1, osem0, osem1):
    idx_v = (idx_v0, idx_v1)
    dists_v = (dists_v0, dists_v1)
    wt_v = (wt_v0, wt_v1)
    dtr_v = (dtr_v0, dtr_v1)
    rows_v = (rows_v0, rows_v1)
    out_v = (out_v0, out_v1)
    isem = (isem0, isem1)
    gsem = (gsem0, gsem1)
    osem = (osem0, osem1)

    cid = lax.axis_index("c")
    sid = lax.axis_index("s")
    wid = sid * NC + cid
    lanes = lax.iota(jnp.int32, 16)
    col8 = lanes * 8
    wb = wid // WPB
    woff = (wid % WPB) * NPW

    def in_copies(i, p):
        cidx = wid * NCHUNK + i
        return (pltpu.make_async_copy(idx_hbm.at[cidx], idx_v[p], isem[p]),
                pltpu.make_async_copy(dists_hbm.at[cidx], dists_v[p],
                                      isem[p]))

    def gathers(p):
        return [pltpu.make_async_copy(feat_hbm.at[idx_v[p].at[g]],
                                      rows_v[p].at[pl.ds(g * 128, 128)],
                                      gsem[p])
                for g in range(8)]

    def out_copies(i, p):
        cidx = wid * NCHUNK + i
        r0 = pl.multiple_of(wb * K, 8)
        c0 = pl.multiple_of(woff + i * Q, 128)
        return (pltpu.make_async_copy(out_v[p], img_hbm.at[cidx], osem[p]),
                pltpu.make_async_copy(dtr_v[p],
                                      dtr_hbm.at[pl.ds(r0, K), pl.ds(c0, Q)],
                                      osem[p]))

    def weights(p):
        for g in range(8):
            gv = jnp.full((16,), g, jnp.int32)
            trans = jnp.full((16,), 1.0, jnp.float32)
            for k in range(K):
                d = plsc.load_gather(dists_v[p], [gv, col8 + k])
                dtr_v[p][k, pl.ds(g * 16, 16)] = d
                alpha = jnp.clip(1.0 - d, 0.0, 1.0)
                wt_v[p][k, pl.ds(g * 16, 16)] = alpha * trans
                trans = trans * (1.0 - alpha)

    def accumulate(p):
        def grp_body(g, carry):
            base = g * 16
            wts = [wt_v[p][k, pl.ds(base, 16)] for k in range(K)]
            for i in range(16):
                acc0 = jnp.zeros((CH,), jnp.float32)
                acc1 = jnp.zeros((CH,), jnp.float32)
                for k in range(K):
                    wk = wts[k][i]
                    r = g * 128 + i * 8 + k
                    acc0 = acc0 + wk * rows_v[p][r, pl.ds(0, CH)]
                    acc1 = acc1 + wk * rows_v[p][r, pl.ds(CH, CH)]
                out_v[p][base + i, pl.ds(0, CH)] = acc0
                out_v[p][base + i, pl.ds(CH, CH)] = acc1
            return carry

        lax.fori_loop(0, 8, grp_body, None)

    def phase(i, p):
        q = 1 - p

        @pl.when(i + 1 < NCHUNK)
        def _():
            for c in in_copies(i + 1, q):
                c.wait()
            for c in gathers(q):
                c.start()

        for c in gathers(p):
            c.wait()

        @pl.when(i >= 2)
        def _():
            for c in out_copies(i - 2, p):
                c.wait()

        weights(p)

        @pl.when(i + 2 < NCHUNK)
        def _():
            for c in in_copies(i + 2, p):
                c.start()

        accumulate(p)
        for c in out_copies(i, p):
            c.start()

    for c in in_copies(0, 0):
        c.start()
    for c in in_copies(0, 0):
        c.wait()
    for c in gathers(0):
        c.start()
    for c in in_copies(1, 1):
        c.start()

    def tbody(t, carry):
        i0 = 2 * t

        phase(i0, 0)

        @pl.when(i0 + 1 < NCHUNK)
        def _():
            phase(i0 + 1, 1)

        return carry

    lax.fori_loop(0, (NCHUNK + 1) // 2, tbody, None)

    for c in out_copies(NCHUNK - 2, (NCHUNK - 2) % 2):
        c.wait()
    for c in out_copies(NCHUNK - 1, (NCHUNK - 1) % 2):
        c.wait()


_sc_render_call = functools.partial(
    pl.kernel,
    out_type=[jax.ShapeDtypeStruct((NW * NCHUNK, Q, C), jnp.float32),
              jax.ShapeDtypeStruct((B * K, HW), jnp.float32)],
    mesh=plsc.VectorSubcoreMesh(core_axis_name="c", subcore_axis_name="s"),
    scratch_types=[
        pltpu.VMEM((K, 128), jnp.int32),
        pltpu.VMEM((K, 128), jnp.int32),
        pltpu.VMEM((K, 128), jnp.float32),
        pltpu.VMEM((K, 128), jnp.float32),
        pltpu.VMEM((K, Q), jnp.float32),
        pltpu.VMEM((K, Q), jnp.float32),
        pltpu.VMEM((K, Q), jnp.float32),
        pltpu.VMEM((K, Q), jnp.float32),
        pltpu.VMEM((Q * K, C), jnp.float32),
        pltpu.VMEM((Q * K, C), jnp.float32),
        pltpu.VMEM((Q, C), jnp.float32),
        pltpu.VMEM((Q, C), jnp.float32),
        pltpu.SemaphoreType.DMA,
        pltpu.SemaphoreType.DMA,
        pltpu.SemaphoreType.DMA,
        pltpu.SemaphoreType.DMA,
        pltpu.SemaphoreType.DMA,
        pltpu.SemaphoreType.DMA,
    ],
    compiler_params=pltpu.CompilerParams(needs_layout_passes=False,
                                         use_tc_tiling_on_sc=False),
)(_sc_render)


def kernel(idx, dists, features):
    idx3 = idx.reshape(NW * NCHUNK, K, 128)
    dists3 = dists.reshape(NW * NCHUNK, K, 128)
    img3, dtr = _sc_render_call(idx3, dists3, features)
    images = img3.reshape(B, H, W, C)
    dists2 = dtr.reshape(B, K, H, W)
    return images, dists2


# speedup vs baseline: 20.5161x; 1.4751x over previous
import functools

import jax
import jax.numpy as jnp
from jax import lax
from jax.experimental import pallas as pl
from jax.experimental.pallas import tpu as pltpu
from jax.experimental.pallas import tpu_sc as plsc

B, H, W, K, P, C = 4, 224, 224, 8, 100000, 32
N = B * H * W
HW = H * W
NC, NS = 2, 16
NW = NC * NS
NPW = N // NW
Q = 128
NCHUNK = NPW // Q
WPB = HW // NPW
CH = C // 2


def _sc_render(idx_hbm, dists_hbm, feat_hbm, img_hbm, dtr_hbm,
               idx_v0, idx_v1, dists_v0, dists_v1, wt_v0, wt_v1,
               dtr_v0, dtr_v1, rows_v0, rows_v1, out_v0, out_v1,
               isem0, isem1, gsem0, gsem1, osem0, osem1):
    idx_v = (idx_v0, idx_v1)
    dists_v = (dists_v0, dists_v1)
    wt_v = (wt_v0, wt_v1)
    dtr_v = (dtr_v0, dtr_v1)
    rows_v = (rows_v0, rows_v1)
    out_v = (out_v0, out_v1)
    isem = (isem0, isem1)
    gsem = (gsem0, gsem1)
    osem = (osem0, osem1)

    cid = lax.axis_index("c")
    sid = lax.axis_index("s")
    wid = sid * NC + cid
    wb = wid // WPB
    woff = (wid % WPB) * NPW
    r0 = pl.multiple_of(wb * K, 8)

    def in_copies(i, p):
        c0 = pl.multiple_of(woff + i * Q, 128)
        return (pltpu.make_async_copy(
                    idx_hbm.at[pl.ds(r0, K), pl.ds(c0, Q)], idx_v[p],
                    isem[p]),
                pltpu.make_async_copy(
                    dists_hbm.at[pl.ds(r0, K), pl.ds(c0, Q)], dists_v[p],
                    isem[p]))

    def gathers(p):
        return [pltpu.make_async_copy(feat_hbm.at[idx_v[p].at[k]],
                                      rows_v[p].at[pl.ds(k * 128, 128)],
                                      gsem[p])
                for k in range(K)]

    def out_copies(i, p):
        cidx = wid * NCHUNK + i
        c0 = pl.multiple_of(woff + i * Q, 128)
        return (pltpu.make_async_copy(out_v[p], img_hbm.at[cidx], osem[p]),
                pltpu.make_async_copy(dtr_v[p],
                                      dtr_hbm.at[pl.ds(r0, K), pl.ds(c0, Q)],
                                      osem[p]))

    def weights(p):
        for g in range(8):
            sl = pl.ds(g * 16, 16)
            trans = jnp.full((16,), 1.0, jnp.float32)
            for k in range(K):
                d = dists_v[p][k, sl]
                dtr_v[p][k, sl] = d
                alpha = jnp.clip(1.0 - d, 0.0, 1.0)
                wt_v[p][k, sl] = alpha * trans
                trans = trans * (1.0 - alpha)

    def accumulate(p):
        def grp_body(g, carry):
            base = g * 16
            wts = [wt_v[p][k, pl.ds(base, 16)] for k in range(K)]
            for i0 in range(0, 16, 2):
                accs = []
                for i in (i0, i0 + 1):
                    wks = [wts[k][i] for k in range(K)]
                    for lo in (0, CH):
                        pr = [wks[k] * rows_v[p][k * 128 + base + i,
                                                 pl.ds(lo, CH)]
                              for k in range(K)]
                        accs.append(((pr[0] + pr[1]) + (pr[2] + pr[3]))
                                    + ((pr[4] + pr[5]) + (pr[6] + pr[7])))
                out_v[p][base + i0, pl.ds(0, CH)] = accs[0]
                out_v[p][base + i0, pl.ds(CH, CH)] = accs[1]
                out_v[p][base + i0 + 1, pl.ds(0, CH)] = accs[2]
                out_v[p][base + i0 + 1, pl.ds(CH, CH)] = accs[3]
            return carry

        lax.fori_loop(0, 8, grp_body, None)

    def phase(i, p):
        q = 1 - p

        @pl.when(i + 1 < NCHUNK)
        def _():
            for c in in_copies(i + 1, q):
                c.wait()
            for c in gathers(q):
                c.start()

        for c in gathers(p):
            c.wait()

        @pl.when(i >= 2)
        def _():
            for c in out_copies(i - 2, p):
                c.wait()

        weights(p)
        accumulate(p)
        for c in out_copies(i, p):
            c.start()

        @pl.when(i + 2 < NCHUNK)
        def _():
            for c in in_copies(i + 2, p):
                c.start()

    for c in in_copies(0, 0):
        c.start()
    for c in in_copies(0, 0):
        c.wait()
    for c in gathers(0):
        c.start()
    for c in in_copies(1, 1):
        c.start()

    def tbody(t, carry):
        i0 = 2 * t

        phase(i0, 0)

        @pl.when(i0 + 1 < NCHUNK)
        def _():
            phase(i0 + 1, 1)

        return carry

    lax.fori_loop(0, (NCHUNK + 1) // 2, tbody, None)

    for c in out_copies(NCHUNK - 2, (NCHUNK - 2) % 2):
        c.wait()
    for c in out_copies(NCHUNK - 1, (NCHUNK - 1) % 2):
        c.wait()


_sc_render_call = functools.partial(
    pl.kernel,
    out_type=[jax.ShapeDtypeStruct((NW * NCHUNK, Q, C), jnp.float32),
              jax.ShapeDtypeStruct((B * K, HW), jnp.float32)],
    mesh=plsc.VectorSubcoreMesh(core_axis_name="c", subcore_axis_name="s"),
    scratch_types=[
        pltpu.VMEM((K, Q), jnp.int32),
        pltpu.VMEM((K, Q), jnp.int32),
        pltpu.VMEM((K, Q), jnp.float32),
        pltpu.VMEM((K, Q), jnp.float32),
        pltpu.VMEM((K, Q), jnp.float32),
        pltpu.VMEM((K, Q), jnp.float32),
        pltpu.VMEM((K, Q), jnp.float32),
        pltpu.VMEM((K, Q), jnp.float32),
        pltpu.VMEM((Q * K, C), jnp.float32),
        pltpu.VMEM((Q * K, C), jnp.float32),
        pltpu.VMEM((Q, C), jnp.float32),
        pltpu.VMEM((Q, C), jnp.float32),
        pltpu.SemaphoreType.DMA,
        pltpu.SemaphoreType.DMA,
        pltpu.SemaphoreType.DMA,
        pltpu.SemaphoreType.DMA,
        pltpu.SemaphoreType.DMA,
        pltpu.SemaphoreType.DMA,
    ],
    compiler_params=pltpu.CompilerParams(needs_layout_passes=False,
                                         use_tc_tiling_on_sc=False),
)(_sc_render)


def kernel(idx, dists, features):
    idx_km = jnp.transpose(idx, (0, 3, 1, 2)).reshape(B * K, HW)
    dists_km = jnp.transpose(dists, (0, 3, 1, 2)).reshape(B * K, HW)
    img3, dtr = _sc_render_call(idx_km, dists_km, features)
    images = img3.reshape(B, H, W, C)
    dists2 = dtr.reshape(B, K, H, W)
    return images, dists2
